# baseline (speedup 1.0000x reference)
import functools

import jax
import jax.numpy as jnp
from jax import lax
from jax.experimental import pallas as pl
from jax.experimental.pallas import tpu as pltpu

_K = 8192
_H = 4096
_D = 32
_BT = 256


def _vq_block(x_ref, w_ref, rs_ref, ws_ref, idx_ref, st_ref, loss_ref):
    x = x_ref[0]
    w = w_ref[...]
    rs = rs_ref[0]
    x2 = (x + x).astype(jnp.bfloat16)
    m = lax.dot_general(w, x2, (((1,), (0,)), ((), ())),
                        preferred_element_type=jnp.float32)
    d = (rs - m) + ws_ref[...]
    dA = d[:_H, :]
    dB = d[_H:, :]
    ksh = lax.broadcasted_iota(jnp.int32, dA.shape, 0)
    mA = jnp.min(dA, axis=0)
    mB = jnp.min(dB, axis=0)
    iA = jnp.min(jnp.where(dA == mA[None, :], ksh, _K), axis=0)
    iB = jnp.min(jnp.where(dB == mB[None, :], ksh, _K), axis=0) + _H
    qA = mA.astype(jnp.bfloat16).astype(jnp.float32)
    idx = jnp.where(qA <= mB, iA, iB)
    idx_ref[...] = idx[None, None, :]
    ks = lax.broadcasted_iota(jnp.int32, d.shape, 0)
    onehot = (ks == idx[None, :]).astype(jnp.bfloat16)
    q = lax.dot_general(w.astype(jnp.bfloat16), onehot,
                        (((0,), (0,)), ((), ())),
                        preferred_element_type=jnp.float32)
    st_ref[0] = x + (q - x)
    part = jnp.sum((q - x) ** 2, keepdims=True)

    @pl.when((pl.program_id(0) == 0) & (pl.program_id(1) == 0))
    def _():
        loss_ref[...] = jnp.zeros_like(loss_ref)

    loss_ref[...] += part


def kernel(z, W):
    bsz, channels, height, width = z.shape
    hw = height * width
    n_tok = bsz * hw
    zc = z.reshape(bsz, channels, hw)
    bpb = hw // _BT
    rowsq = jnp.sum(z ** 2, axis=1).reshape(bsz * bpb, 1, _BT)
    wsq = jnp.sum(W ** 2, axis=1)[:, None]
    idx, st, loss_sum = pl.pallas_call(
        _vq_block,
        grid=(bsz, bpb),
        in_specs=[
            pl.BlockSpec((1, _D, _BT), lambda b, j: (b, 0, j)),
            pl.BlockSpec((_K, _D), lambda b, j: (0, 0)),
            pl.BlockSpec((1, 1, _BT), lambda b, j: (b * bpb + j, 0, 0)),
            pl.BlockSpec((_K, 1), lambda b, j: (0, 0)),
        ],
        out_specs=[
            pl.BlockSpec((1, 1, _BT), lambda b, j: (b * bpb + j, 0, 0)),
            pl.BlockSpec((1, _D, _BT), lambda b, j: (b, 0, j)),
            pl.BlockSpec((1, 1), lambda b, j: (0, 0)),
        ],
        out_shape=[
            jax.ShapeDtypeStruct((bsz * bpb, 1, _BT), jnp.int32),
            jax.ShapeDtypeStruct((bsz, _D, hw), jnp.float32),
            jax.ShapeDtypeStruct((1, 1), jnp.float32),
        ],
    )(zc, W, rowsq, wsq)
    quantized_st = st.reshape(bsz, channels, height, width)
    codebook_loss = loss_sum[0, 0] / (n_tok * channels)
    commitment_loss = 0.25 * codebook_loss
    indices = idx.reshape(bsz, height, width)
    return quantized_st, codebook_loss, commitment_loss, indices


# speedup vs baseline: 1.2411x; 1.2411x over previous
import functools

import jax
import jax.numpy as jnp
from jax import lax
from jax.experimental import pallas as pl
from jax.experimental.pallas import tpu as pltpu

_K = 8192
_H = 4096
_D = 32
_BT = 512


def _vq_block(x_ref, w_ref, rs_ref, ws_ref, idx_ref, st_ref, loss_ref):
    x = x_ref[0]
    w = w_ref[...]
    rs = rs_ref[0]
    x2 = (x + x).astype(jnp.bfloat16)
    m = lax.dot_general(w, x2, (((1,), (0,)), ((), ())),
                        preferred_element_type=jnp.float32)
    d = (rs - m) + ws_ref[...]
    dA = d[:_H, :]
    dB = d[_H:, :]
    ksh = lax.broadcasted_iota(jnp.int32, dA.shape, 0)
    mA = jnp.min(dA, axis=0)
    mB = jnp.min(dB, axis=0)
    iA = jnp.min(jnp.where(dA == mA[None, :], ksh, _K), axis=0)
    iB = jnp.min(jnp.where(dB == mB[None, :], ksh, _K), axis=0) + _H
    qA = mA.astype(jnp.bfloat16).astype(jnp.float32)
    idx = jnp.where(qA <= mB, iA, iB)
    idx_ref[...] = idx[None, None, :]
    ks = lax.broadcasted_iota(jnp.int32, d.shape, 0)
    onehot = (ks == idx[None, :]).astype(jnp.bfloat16)
    q = lax.dot_general(w.astype(jnp.bfloat16), onehot,
                        (((0,), (0,)), ((), ())),
                        preferred_element_type=jnp.float32)
    st_ref[0] = x + (q - x)
    part = jnp.sum((q - x) ** 2, keepdims=True)

    @pl.when((pl.program_id(0) == 0) & (pl.program_id(1) == 0))
    def _():
        loss_ref[...] = jnp.zeros_like(loss_ref)

    loss_ref[...] += part


def kernel(z, W):
    bsz, channels, height, width = z.shape
    hw = height * width
    n_tok = bsz * hw
    zc = z.reshape(bsz, channels, hw)
    bpb = hw // _BT
    rowsq = jnp.sum(z ** 2, axis=1).reshape(bsz * bpb, 1, _BT)
    wsq = jnp.sum(W ** 2, axis=1)[:, None]
    idx, st, loss_sum = pl.pallas_call(
        _vq_block,
        grid=(bsz, bpb),
        in_specs=[
            pl.BlockSpec((1, _D, _BT), lambda b, j: (b, 0, j)),
            pl.BlockSpec((_K, _D), lambda b, j: (0, 0)),
            pl.BlockSpec((1, 1, _BT), lambda b, j: (b * bpb + j, 0, 0)),
            pl.BlockSpec((_K, 1), lambda b, j: (0, 0)),
        ],
        out_specs=[
            pl.BlockSpec((1, 1, _BT), lambda b, j: (b * bpb + j, 0, 0)),
            pl.BlockSpec((1, _D, _BT), lambda b, j: (b, 0, j)),
            pl.BlockSpec((1, 1), lambda b, j: (0, 0)),
        ],
        out_shape=[
            jax.ShapeDtypeStruct((bsz * bpb, 1, _BT), jnp.int32),
            jax.ShapeDtypeStruct((bsz, _D, hw), jnp.float32),
            jax.ShapeDtypeStruct((1, 1), jnp.float32),
        ],
    )(zc, W, rowsq, wsq)
    quantized_st = st.reshape(bsz, channels, height, width)
    codebook_loss = loss_sum[0, 0] / (n_tok * channels)
    commitment_loss = 0.25 * codebook_loss
    indices = idx.reshape(bsz, height, width)
    return quantized_st, codebook_loss, commitment_loss, indices
